# baseline (speedup 1.0000x reference)
import jax
import jax.numpy as jnp
from jax.experimental import pallas as pl
from jax.experimental.pallas import tpu as pltpu

N = 16384
D = 2048
BLK = 256


def _body(t_ref, s_ref, xp_ref, x_ref, o_ref):
    d = xp_ref[...] - x_ref[...]
    ssq = jnp.sum(d * d, axis=1)
    s = s_ref[0, 0]
    logs = jnp.log(s)
    scale = -logs * jnp.exp(-2.0 * logs * t_ref[:, 0])
    o_ref[...] = scale * ssq


def kernel(t, sigma1, x_pred, x):
    n, d = x.shape
    s2d = sigma1.reshape(1, 1)
    grid = (n // BLK,)
    out = pl.pallas_call(
        _body,
        grid=grid,
        in_specs=[
            pl.BlockSpec((BLK, 1), lambda i: (i, 0)),
            pl.BlockSpec((1, 1), lambda i: (0, 0)),
            pl.BlockSpec((BLK, d), lambda i: (i, 0)),
            pl.BlockSpec((BLK, d), lambda i: (i, 0)),
        ],
        out_specs=pl.BlockSpec((BLK,), lambda i: (i,)),
        out_shape=jax.ShapeDtypeStruct((n,), jnp.float32),
        compiler_params=pltpu.CompilerParams(
            dimension_semantics=("arbitrary",),
        ),
    )(t, s2d, x_pred, x)
    return out


# speedup vs baseline: 1.0410x; 1.0410x over previous
import jax
import jax.numpy as jnp
from jax.experimental import pallas as pl
from jax.experimental.pallas import tpu as pltpu

N = 16384
D = 2048
BLK = 512


def _body(t_ref, s_ref, xp_ref, x_ref, o_ref):
    d = xp_ref[...] - x_ref[...]
    d2 = d * d
    ones = jnp.ones((d2.shape[1], 128), jnp.float32)
    partial = jax.lax.dot_general(
        d2, ones, (((1,), (0,)), ((), ())),
        preferred_element_type=jnp.float32,
    )
    ssq = partial[:, 0]
    s = s_ref[0, 0]
    logs = jnp.log(s)
    scale = -logs * jnp.exp(-2.0 * logs * t_ref[:, 0])
    o_ref[...] = scale * ssq


def kernel(t, sigma1, x_pred, x):
    n, d = x.shape
    s2d = sigma1.reshape(1, 1)
    grid = (n // BLK,)
    out = pl.pallas_call(
        _body,
        grid=grid,
        in_specs=[
            pl.BlockSpec((BLK, 1), lambda i: (i, 0)),
            pl.BlockSpec((1, 1), lambda i: (0, 0)),
            pl.BlockSpec((BLK, d), lambda i: (i, 0)),
            pl.BlockSpec((BLK, d), lambda i: (i, 0)),
        ],
        out_specs=pl.BlockSpec((BLK,), lambda i: (i,)),
        out_shape=jax.ShapeDtypeStruct((n,), jnp.float32),
        compiler_params=pltpu.CompilerParams(
            dimension_semantics=("arbitrary",),
        ),
    )(t, s2d, x_pred, x)
    return out


# speedup vs baseline: 1.1285x; 1.0841x over previous
import jax
import jax.numpy as jnp
from jax.experimental import pallas as pl
from jax.experimental.pallas import tpu as pltpu

N = 16384
D = 2048
BLK = 512


def _body(t_ref, s_ref, xp_ref, x_ref, o_ref):
    d = xp_ref[...] - x_ref[...]
    ssq = jnp.sum(d * d, axis=1)
    s = s_ref[0, 0]
    logs = jnp.log(s)
    scale = -logs * jnp.exp(-2.0 * logs * t_ref[:, 0])
    o_ref[...] = scale * ssq


def kernel(t, sigma1, x_pred, x):
    n, d = x.shape
    s2d = sigma1.reshape(1, 1)
    grid = (n // BLK,)
    out = pl.pallas_call(
        _body,
        grid=grid,
        in_specs=[
            pl.BlockSpec((BLK, 1), lambda i: (i, 0)),
            pl.BlockSpec((1, 1), lambda i: (0, 0)),
            pl.BlockSpec((BLK, d), lambda i: (i, 0)),
            pl.BlockSpec((BLK, d), lambda i: (i, 0)),
        ],
        out_specs=pl.BlockSpec((BLK,), lambda i: (i,)),
        out_shape=jax.ShapeDtypeStruct((n,), jnp.float32),
        compiler_params=pltpu.CompilerParams(
            dimension_semantics=("arbitrary",),
        ),
    )(t, s2d, x_pred, x)
    return out


# speedup vs baseline: 1.2365x; 1.0957x over previous
import jax
import jax.numpy as jnp
from jax.experimental import pallas as pl
from jax.experimental.pallas import tpu as pltpu

N = 16384
D = 2048
BLK = 512


def _body(t_ref, s_ref, xp_ref, x_ref, o_ref):
    blk, dd = xp_ref.shape
    R = 128
    ones_row = jnp.ones((1, 128), jnp.float32)
    s = s_ref[0, 0]
    logs = jnp.log(s)
    for c in range(blk // R):
        r0 = c * R
        acc = jnp.zeros((R, 128), jnp.float32)
        for k in range(dd // 128):
            dk = (xp_ref[r0:r0 + R, k * 128:(k + 1) * 128]
                  - x_ref[r0:r0 + R, k * 128:(k + 1) * 128])
            acc = acc + dk * dk
        ssq = jax.lax.dot_general(
            ones_row, acc, (((1,), (1,)), ((), ())),
            preferred_element_type=jnp.float32,
        )
        scale = -logs * jnp.exp(-2.0 * logs * t_ref[0, :, r0:r0 + R])
        o_ref[0, :, r0:r0 + R] = scale * ssq


def kernel(t, sigma1, x_pred, x):
    n, d = x.shape
    s2d = sigma1.reshape(1, 1)
    t2d = t.reshape(n // BLK, 1, BLK)
    grid = (n // BLK,)
    out = pl.pallas_call(
        _body,
        grid=grid,
        in_specs=[
            pl.BlockSpec((1, 1, BLK), lambda i: (i, 0, 0)),
            pl.BlockSpec((1, 1), lambda i: (0, 0)),
            pl.BlockSpec((BLK, d), lambda i: (i, 0)),
            pl.BlockSpec((BLK, d), lambda i: (i, 0)),
        ],
        out_specs=pl.BlockSpec((1, 1, BLK), lambda i: (i, 0, 0)),
        out_shape=jax.ShapeDtypeStruct((n // BLK, 1, BLK), jnp.float32),
        compiler_params=pltpu.CompilerParams(
            dimension_semantics=("arbitrary",),
        ),
    )(t2d, s2d, x_pred, x)
    return out.reshape(n)
